# baseline (speedup 1.0000x reference)
import jax
import jax.numpy as jnp
import numpy as np
from jax.experimental import pallas as pl
from jax.experimental.pallas import tpu as pltpu

BS, NUM_PATCH, N_VARS, PATCH_LEN = 64, 64, 7, 16
D_MODEL, D_FF, N_EXPERTS, TOP_K = 128, 256, 10, 2
ROWS_PER_BATCH = NUM_PATCH * N_VARS
BATCHES_PER_TILE = 2
TILE = ROWS_PER_BATCH * BATCHES_PER_TILE


def _sincos_pos(num_patch, d_model):
    pos = np.arange(num_patch)[:, None].astype(np.float64)
    i = np.arange(d_model)[None, :].astype(np.float64)
    angle = pos / np.power(10000.0, (2.0 * (i // 2)) / d_model)
    pe = np.zeros((num_patch, d_model), dtype=np.float32)
    pe[:, 0::2] = np.sin(angle[:, 0::2])
    pe[:, 1::2] = np.cos(angle[:, 1::2])
    return pe


def _moe_block(zt_ref, pe_ref, W_emb_ref, b_emb_ref, W_router_ref,
               W1c_ref, b1c_ref, W2c_ref, b2_ref, W_head_ref, b_head_ref,
               y_ref):
    x = jnp.dot(zt_ref[...].reshape(TILE, PATCH_LEN), W_emb_ref[...],
                preferred_element_type=jnp.float32)
    x = x + b_emb_ref[...] + pe_ref[...]

    logits = jnp.dot(x, W_router_ref[...], preferred_element_type=jnp.float32)
    probs = jax.nn.softmax(logits, axis=-1)
    e_ids = jax.lax.broadcasted_iota(jnp.int32, (TILE, N_EXPERTS), 1)
    v1 = jnp.max(probs, axis=-1, keepdims=True)
    i1 = jnp.min(jnp.where(probs == v1, e_ids, N_EXPERTS), axis=-1,
                 keepdims=True)
    m1 = e_ids == i1
    probs2 = jnp.where(m1, -1.0, probs)
    v2 = jnp.max(probs2, axis=-1, keepdims=True)
    i2 = jnp.min(jnp.where(probs2 == v2, e_ids, N_EXPERTS), axis=-1,
                 keepdims=True)
    m2 = e_ids == i2
    denom = v1 + v2
    gates = jnp.where(m1, v1 / denom, 0.0) + jnp.where(m2, v2 / denom, 0.0)

    xb = x.astype(jnp.bfloat16)
    hb = jnp.dot(xb, W1c_ref[...],
                 preferred_element_type=jnp.float32).astype(jnp.bfloat16)
    hb = hb + b1c_ref[...]
    cc = jnp.bfloat16(0.7978845608028654)
    ca = jnp.bfloat16(0.035677408136300153)
    u2 = hb * hb
    z = hb * (u2 * ca + cc)
    t = jnp.tanh(z)
    Hb = hb + hb * t
    gb = (0.5 * gates).astype(jnp.bfloat16)
    Hg = jnp.concatenate(
        [Hb[:, e * D_FF:(e + 1) * D_FF] * gb[:, e:e + 1]
         for e in range(N_EXPERTS)], axis=1)
    acc = jnp.dot(Hg, W2c_ref[...], preferred_element_type=jnp.float32)
    acc = acc + jnp.dot(gates, b2_ref[...],
                        preferred_element_type=jnp.float32)

    tokens = x + acc
    y = jnp.dot(tokens, W_head_ref[...],
                preferred_element_type=jnp.float32) + b_head_ref[...]
    y_ref[...] = y.reshape(1, TILE, PATCH_LEN)


@jax.jit
def kernel(z, W_emb, b_emb, W_router, W1, b1, W2, b2, W_head, b_head):
    bs, num_patch, n_vars, patch_len = z.shape
    d_model = W_emb.shape[1]
    n_tiles = bs // BATCHES_PER_TILE
    zt = z.reshape(n_tiles, TILE, patch_len)
    pe = _sincos_pos(num_patch, d_model)
    pe_tile = jnp.asarray(
        np.tile(np.repeat(pe, n_vars, axis=0), (BATCHES_PER_TILE, 1)))

    y_flat = pl.pallas_call(
        _moe_block,
        grid=(n_tiles,),
        in_specs=[
            pl.BlockSpec((1, TILE, patch_len), lambda i: (i, 0, 0)),
            pl.BlockSpec((TILE, d_model), lambda i: (0, 0)),
            pl.BlockSpec((patch_len, d_model), lambda i: (0, 0)),
            pl.BlockSpec((d_model,), lambda i: (0,)),
            pl.BlockSpec((d_model, N_EXPERTS), lambda i: (0, 0)),
            pl.BlockSpec((d_model, N_EXPERTS * D_FF), lambda i: (0, 0)),
            pl.BlockSpec((N_EXPERTS * D_FF,), lambda i: (0,)),
            pl.BlockSpec((N_EXPERTS * D_FF, d_model), lambda i: (0, 0)),
            pl.BlockSpec((N_EXPERTS, d_model), lambda i: (0, 0)),
            pl.BlockSpec((d_model, patch_len), lambda i: (0, 0)),
            pl.BlockSpec((patch_len,), lambda i: (0,)),
        ],
        out_specs=pl.BlockSpec((1, TILE, patch_len), lambda i: (i, 0, 0)),
        out_shape=jax.ShapeDtypeStruct((n_tiles, TILE, patch_len),
                                       jnp.float32),
        compiler_params=pltpu.CompilerParams(
            dimension_semantics=("arbitrary",)),
    )(zt, pe_tile, W_emb, b_emb, W_router,
      W1.transpose(1, 0, 2).reshape(d_model, N_EXPERTS * D_FF)
        .astype(jnp.bfloat16),
      b1.reshape(N_EXPERTS * D_FF).astype(jnp.bfloat16),
      W2.reshape(N_EXPERTS * D_FF, d_model).astype(jnp.bfloat16),
      b2, W_head, b_head)

    y = y_flat.reshape(bs, num_patch, n_vars, patch_len)
    y = y.transpose(0, 1, 3, 2).reshape(bs, num_patch * patch_len, n_vars)
    return y


# speedup vs baseline: 1.1762x; 1.1762x over previous
import jax
import jax.numpy as jnp
import numpy as np
from jax.experimental import pallas as pl
from jax.experimental.pallas import tpu as pltpu

BS, NUM_PATCH, N_VARS, PATCH_LEN = 64, 64, 7, 16
D_MODEL, D_FF, N_EXPERTS, TOP_K = 128, 256, 10, 2
TOKENS = BS * N_VARS * NUM_PATCH
TILE = 1024


def _sincos_pos(num_patch, d_model):
    pos = np.arange(num_patch)[:, None].astype(np.float64)
    i = np.arange(d_model)[None, :].astype(np.float64)
    angle = pos / np.power(10000.0, (2.0 * (i // 2)) / d_model)
    pe = np.zeros((num_patch, d_model), dtype=np.float32)
    pe[:, 0::2] = np.sin(angle[:, 0::2])
    pe[:, 1::2] = np.cos(angle[:, 1::2])
    return pe


def _moe_block(zt_ref, pe_ref, W_emb_ref, b_emb_ref, W_router_ref,
               W1c_ref, b1c_ref, W2c_ref, b2_ref, W_head_ref, b_head_ref,
               y_ref):
    x = jnp.dot(zt_ref[...], W_emb_ref[...],
                preferred_element_type=jnp.float32)
    x = x + b_emb_ref[...] + pe_ref[...]

    logits = jnp.dot(x, W_router_ref[...], preferred_element_type=jnp.float32)
    e_ids = jax.lax.broadcasted_iota(jnp.int32, (TILE, N_EXPERTS), 1)
    v1 = jnp.max(logits, axis=-1, keepdims=True)
    i1 = jnp.min(jnp.where(logits == v1, e_ids, N_EXPERTS), axis=-1,
                 keepdims=True)
    m1 = e_ids == i1
    logits2 = jnp.where(m1, -jnp.inf, logits)
    v2 = jnp.max(logits2, axis=-1, keepdims=True)
    i2 = jnp.min(jnp.where(logits2 == v2, e_ids, N_EXPERTS), axis=-1,
                 keepdims=True)
    m2 = e_ids == i2
    w1 = 1.0 / (1.0 + jnp.exp(v2 - v1))
    gates = jnp.where(m1, w1, 0.0) + jnp.where(m2, 1.0 - w1, 0.0)

    xb = x.astype(jnp.bfloat16)
    hb = jnp.dot(xb, W1c_ref[...],
                 preferred_element_type=jnp.float32).astype(jnp.bfloat16)
    hb = hb + b1c_ref[...]
    cc = jnp.bfloat16(0.7978845608028654)
    ca = jnp.bfloat16(0.035677408136300153)
    u2 = hb * hb
    z = hb * (u2 * ca + cc)
    t = jnp.tanh(z)
    Hb = hb + hb * t
    gb = (0.5 * gates).astype(jnp.bfloat16)
    Hg = jnp.concatenate(
        [Hb[:, e * D_FF:(e + 1) * D_FF] * gb[:, e:e + 1]
         for e in range(N_EXPERTS)], axis=1)
    acc = jnp.dot(Hg, W2c_ref[...], preferred_element_type=jnp.float32)
    acc = acc + jnp.dot(gates, b2_ref[...],
                        preferred_element_type=jnp.float32)

    tokens = x + acc
    y_ref[...] = jnp.dot(tokens, W_head_ref[...],
                         preferred_element_type=jnp.float32) + b_head_ref[...]


@jax.jit
def kernel(z, W_emb, b_emb, W_router, W1, b1, W2, b2, W_head, b_head):
    bs, num_patch, n_vars, patch_len = z.shape
    d_model = W_emb.shape[1]
    zt = z.transpose(0, 2, 1, 3).reshape(TOKENS, patch_len)
    pe = _sincos_pos(num_patch, d_model)
    pe_tile = jnp.asarray(np.tile(pe, (TILE // num_patch, 1)))

    grid = (TOKENS // TILE,)
    y_flat = pl.pallas_call(
        _moe_block,
        grid=grid,
        in_specs=[
            pl.BlockSpec((TILE, patch_len), lambda i: (i, 0)),
            pl.BlockSpec((TILE, d_model), lambda i: (0, 0)),
            pl.BlockSpec((patch_len, d_model), lambda i: (0, 0)),
            pl.BlockSpec((d_model,), lambda i: (0,)),
            pl.BlockSpec((d_model, N_EXPERTS), lambda i: (0, 0)),
            pl.BlockSpec((d_model, N_EXPERTS * D_FF), lambda i: (0, 0)),
            pl.BlockSpec((N_EXPERTS * D_FF,), lambda i: (0,)),
            pl.BlockSpec((N_EXPERTS * D_FF, d_model), lambda i: (0, 0)),
            pl.BlockSpec((N_EXPERTS, d_model), lambda i: (0, 0)),
            pl.BlockSpec((d_model, patch_len), lambda i: (0, 0)),
            pl.BlockSpec((patch_len,), lambda i: (0,)),
        ],
        out_specs=pl.BlockSpec((TILE, patch_len), lambda i: (i, 0)),
        out_shape=jax.ShapeDtypeStruct((TOKENS, patch_len), jnp.float32),
        compiler_params=pltpu.CompilerParams(
            dimension_semantics=("arbitrary",)),
    )(zt, pe_tile, W_emb, b_emb, W_router,
      W1.transpose(1, 0, 2).reshape(d_model, N_EXPERTS * D_FF)
        .astype(jnp.bfloat16),
      b1.reshape(N_EXPERTS * D_FF).astype(jnp.bfloat16),
      W2.reshape(N_EXPERTS * D_FF, d_model).astype(jnp.bfloat16),
      b2, W_head, b_head)

    y = y_flat.reshape(bs, n_vars, num_patch, patch_len)
    y = y.transpose(0, 2, 3, 1).reshape(bs, num_patch * patch_len, n_vars)
    return y


# speedup vs baseline: 1.2089x; 1.0278x over previous
import jax
import jax.numpy as jnp
import numpy as np
from jax.experimental import pallas as pl
from jax.experimental.pallas import tpu as pltpu

BS, NUM_PATCH, N_VARS, PATCH_LEN = 64, 64, 7, 16
D_MODEL, D_FF, N_EXPERTS, TOP_K = 128, 256, 10, 2
TOKENS = BS * N_VARS * NUM_PATCH
TILE = 1024


def _sincos_pos(num_patch, d_model):
    pos = np.arange(num_patch)[:, None].astype(np.float64)
    i = np.arange(d_model)[None, :].astype(np.float64)
    angle = pos / np.power(10000.0, (2.0 * (i // 2)) / d_model)
    pe = np.zeros((num_patch, d_model), dtype=np.float32)
    pe[:, 0::2] = np.sin(angle[:, 0::2])
    pe[:, 1::2] = np.cos(angle[:, 1::2])
    return pe


def _moe_block(zt_ref, pe_ref, W_emb_ref, b_emb_ref, W_router_ref,
               W1c_ref, b1c_ref, W2c_ref, b2_ref, W_head_ref, b_head_ref,
               y_ref):
    x = jnp.dot(zt_ref[...], W_emb_ref[...],
                preferred_element_type=jnp.float32)
    x = x + b_emb_ref[...] + pe_ref[...]

    logits = jnp.dot(x, W_router_ref[...], preferred_element_type=jnp.float32)
    probs = jax.nn.softmax(logits, axis=-1)
    e_ids = jax.lax.broadcasted_iota(jnp.int32, (TILE, N_EXPERTS), 1)
    v1 = jnp.max(probs, axis=-1, keepdims=True)
    i1 = jnp.min(jnp.where(probs == v1, e_ids, N_EXPERTS), axis=-1,
                 keepdims=True)
    m1 = e_ids == i1
    probs2 = jnp.where(m1, -1.0, probs)
    v2 = jnp.max(probs2, axis=-1, keepdims=True)
    i2 = jnp.min(jnp.where(probs2 == v2, e_ids, N_EXPERTS), axis=-1,
                 keepdims=True)
    m2 = e_ids == i2
    denom = v1 + v2
    gates = jnp.where(m1, v1 / denom, 0.0) + jnp.where(m2, v2 / denom, 0.0)

    xb = x.astype(jnp.bfloat16)
    cc = jnp.bfloat16(0.7978845608028654)
    ca = jnp.bfloat16(0.035677408136300153)
    gb = (0.5 * gates).astype(jnp.bfloat16)
    acc = jnp.dot(gates, b2_ref[...], preferred_element_type=jnp.float32)
    NCHUNK = 2
    EPC = N_EXPERTS // NCHUNK
    W = EPC * D_FF
    for c in range(NCHUNK):
        sl = pl.ds(c * W, W)
        hb = jnp.dot(xb, W1c_ref[:, sl],
                     preferred_element_type=jnp.float32).astype(jnp.bfloat16)
        hb = hb + b1c_ref[sl]
        u2 = hb * hb
        z = hb * (u2 * ca + cc)
        t = jnp.tanh(z)
        Hb = hb + hb * t
        Hg = jnp.concatenate(
            [Hb[:, e * D_FF:(e + 1) * D_FF] * gb[:, c * EPC + e:c * EPC + e + 1]
             for e in range(EPC)], axis=1)
        acc = acc + jnp.dot(Hg, W2c_ref[sl, :],
                            preferred_element_type=jnp.float32)

    tokens = x + acc
    y_ref[...] = jnp.dot(tokens, W_head_ref[...],
                         preferred_element_type=jnp.float32) + b_head_ref[...]


@jax.jit
def kernel(z, W_emb, b_emb, W_router, W1, b1, W2, b2, W_head, b_head):
    bs, num_patch, n_vars, patch_len = z.shape
    d_model = W_emb.shape[1]
    zt = z.transpose(0, 2, 1, 3).reshape(TOKENS, patch_len)
    pe = _sincos_pos(num_patch, d_model)
    pe_tile = jnp.asarray(np.tile(pe, (TILE // num_patch, 1)))

    grid = (TOKENS // TILE,)
    y_flat = pl.pallas_call(
        _moe_block,
        grid=grid,
        in_specs=[
            pl.BlockSpec((TILE, patch_len), lambda i: (i, 0)),
            pl.BlockSpec((TILE, d_model), lambda i: (0, 0)),
            pl.BlockSpec((patch_len, d_model), lambda i: (0, 0)),
            pl.BlockSpec((d_model,), lambda i: (0,)),
            pl.BlockSpec((d_model, N_EXPERTS), lambda i: (0, 0)),
            pl.BlockSpec((d_model, N_EXPERTS * D_FF), lambda i: (0, 0)),
            pl.BlockSpec((N_EXPERTS * D_FF,), lambda i: (0,)),
            pl.BlockSpec((N_EXPERTS * D_FF, d_model), lambda i: (0, 0)),
            pl.BlockSpec((N_EXPERTS, d_model), lambda i: (0, 0)),
            pl.BlockSpec((d_model, patch_len), lambda i: (0, 0)),
            pl.BlockSpec((patch_len,), lambda i: (0,)),
        ],
        out_specs=pl.BlockSpec((TILE, patch_len), lambda i: (i, 0)),
        out_shape=jax.ShapeDtypeStruct((TOKENS, patch_len), jnp.float32),
        compiler_params=pltpu.CompilerParams(
            dimension_semantics=("arbitrary",)),
    )(zt, pe_tile, W_emb, b_emb, W_router,
      W1.transpose(1, 0, 2).reshape(d_model, N_EXPERTS * D_FF)
        .astype(jnp.bfloat16),
      b1.reshape(N_EXPERTS * D_FF).astype(jnp.bfloat16),
      W2.reshape(N_EXPERTS * D_FF, d_model).astype(jnp.bfloat16),
      b2, W_head, b_head)

    y = y_flat.reshape(bs, n_vars, num_patch, patch_len)
    y = y.transpose(0, 2, 3, 1).reshape(bs, num_patch * patch_len, n_vars)
    return y


# speedup vs baseline: 1.2469x; 1.0315x over previous
import jax
import jax.numpy as jnp
import numpy as np
from jax.experimental import pallas as pl
from jax.experimental.pallas import tpu as pltpu

BS, NUM_PATCH, N_VARS, PATCH_LEN = 64, 64, 7, 16
D_MODEL, D_FF, N_EXPERTS, TOP_K = 128, 256, 10, 2
TOKENS = BS * N_VARS * NUM_PATCH
TILE = 2048


def _sincos_pos(num_patch, d_model):
    pos = np.arange(num_patch)[:, None].astype(np.float64)
    i = np.arange(d_model)[None, :].astype(np.float64)
    angle = pos / np.power(10000.0, (2.0 * (i // 2)) / d_model)
    pe = np.zeros((num_patch, d_model), dtype=np.float32)
    pe[:, 0::2] = np.sin(angle[:, 0::2])
    pe[:, 1::2] = np.cos(angle[:, 1::2])
    return pe


def _moe_block(zt_ref, pe_ref, W_emb_ref, b_emb_ref, W_router_ref,
               W1c_ref, b1c_ref, W2c_ref, b2_ref, W_head_ref, b_head_ref,
               y_ref):
    x = jnp.dot(zt_ref[...], W_emb_ref[...],
                preferred_element_type=jnp.float32)
    x = x + b_emb_ref[...] + pe_ref[...]

    logits = jnp.dot(x, W_router_ref[...], preferred_element_type=jnp.float32)
    probs = jax.nn.softmax(logits, axis=-1)
    e_ids = jax.lax.broadcasted_iota(jnp.int32, (TILE, N_EXPERTS), 1)
    v1 = jnp.max(probs, axis=-1, keepdims=True)
    i1 = jnp.min(jnp.where(probs == v1, e_ids, N_EXPERTS), axis=-1,
                 keepdims=True)
    m1 = e_ids == i1
    probs2 = jnp.where(m1, -1.0, probs)
    v2 = jnp.max(probs2, axis=-1, keepdims=True)
    i2 = jnp.min(jnp.where(probs2 == v2, e_ids, N_EXPERTS), axis=-1,
                 keepdims=True)
    m2 = e_ids == i2
    denom = v1 + v2
    gates = jnp.where(m1, v1 / denom, 0.0) + jnp.where(m2, v2 / denom, 0.0)

    xb = x.astype(jnp.bfloat16)
    cc = jnp.bfloat16(0.7978845608028654)
    ca = jnp.bfloat16(0.035677408136300153)
    gb = (0.5 * gates).astype(jnp.bfloat16)
    acc = jnp.dot(gates, b2_ref[...], preferred_element_type=jnp.float32)
    NCHUNK = 2
    EPC = N_EXPERTS // NCHUNK
    W = EPC * D_FF
    for c in range(NCHUNK):
        sl = pl.ds(c * W, W)
        hb = jnp.dot(xb, W1c_ref[:, sl],
                     preferred_element_type=jnp.float32).astype(jnp.bfloat16)
        hb = hb + b1c_ref[sl]
        u2 = hb * hb
        z = hb * (u2 * ca + cc)
        t = jnp.tanh(z)
        Hb = hb + hb * t
        Hg = jnp.concatenate(
            [Hb[:, e * D_FF:(e + 1) * D_FF] * gb[:, c * EPC + e:c * EPC + e + 1]
             for e in range(EPC)], axis=1)
        acc = acc + jnp.dot(Hg, W2c_ref[sl, :],
                            preferred_element_type=jnp.float32)

    tokens = x + acc
    y_ref[...] = jnp.dot(tokens, W_head_ref[...],
                         preferred_element_type=jnp.float32) + b_head_ref[...]


@jax.jit
def kernel(z, W_emb, b_emb, W_router, W1, b1, W2, b2, W_head, b_head):
    bs, num_patch, n_vars, patch_len = z.shape
    d_model = W_emb.shape[1]
    zt = z.transpose(0, 2, 1, 3).reshape(TOKENS, patch_len)
    pe = _sincos_pos(num_patch, d_model)
    pe_tile = jnp.asarray(np.tile(pe, (TILE // num_patch, 1)))

    grid = (TOKENS // TILE,)
    y_flat = pl.pallas_call(
        _moe_block,
        grid=grid,
        in_specs=[
            pl.BlockSpec((TILE, patch_len), lambda i: (i, 0)),
            pl.BlockSpec((TILE, d_model), lambda i: (0, 0)),
            pl.BlockSpec((patch_len, d_model), lambda i: (0, 0)),
            pl.BlockSpec((d_model,), lambda i: (0,)),
            pl.BlockSpec((d_model, N_EXPERTS), lambda i: (0, 0)),
            pl.BlockSpec((d_model, N_EXPERTS * D_FF), lambda i: (0, 0)),
            pl.BlockSpec((N_EXPERTS * D_FF,), lambda i: (0,)),
            pl.BlockSpec((N_EXPERTS * D_FF, d_model), lambda i: (0, 0)),
            pl.BlockSpec((N_EXPERTS, d_model), lambda i: (0, 0)),
            pl.BlockSpec((d_model, patch_len), lambda i: (0, 0)),
            pl.BlockSpec((patch_len,), lambda i: (0,)),
        ],
        out_specs=pl.BlockSpec((TILE, patch_len), lambda i: (i, 0)),
        out_shape=jax.ShapeDtypeStruct((TOKENS, patch_len), jnp.float32),
        compiler_params=pltpu.CompilerParams(
            dimension_semantics=("arbitrary",)),
    )(zt, pe_tile, W_emb, b_emb, W_router,
      W1.transpose(1, 0, 2).reshape(d_model, N_EXPERTS * D_FF)
        .astype(jnp.bfloat16),
      b1.reshape(N_EXPERTS * D_FF).astype(jnp.bfloat16),
      W2.reshape(N_EXPERTS * D_FF, d_model).astype(jnp.bfloat16),
      b2, W_head, b_head)

    y = y_flat.reshape(bs, n_vars, num_patch, patch_len)
    y = y.transpose(0, 2, 3, 1).reshape(bs, num_patch * patch_len, n_vars)
    return y


# speedup vs baseline: 1.2825x; 1.0285x over previous
import jax
import jax.numpy as jnp
import numpy as np
from jax.experimental import pallas as pl
from jax.experimental.pallas import tpu as pltpu

BS, NUM_PATCH, N_VARS, PATCH_LEN = 64, 64, 7, 16
D_MODEL, D_FF, N_EXPERTS, TOP_K = 128, 256, 10, 2
TOKENS = BS * N_VARS * NUM_PATCH
TILE = 4096


def _sincos_pos(num_patch, d_model):
    pos = np.arange(num_patch)[:, None].astype(np.float64)
    i = np.arange(d_model)[None, :].astype(np.float64)
    angle = pos / np.power(10000.0, (2.0 * (i // 2)) / d_model)
    pe = np.zeros((num_patch, d_model), dtype=np.float32)
    pe[:, 0::2] = np.sin(angle[:, 0::2])
    pe[:, 1::2] = np.cos(angle[:, 1::2])
    return pe


def _moe_block(zt_ref, pe_ref, W_emb_ref, b_emb_ref, W_router_ref,
               W1c_ref, b1c_ref, W2c_ref, b2_ref, W_head_ref, b_head_ref,
               y_ref):
    x = jnp.dot(zt_ref[...], W_emb_ref[...],
                preferred_element_type=jnp.float32)
    x = x + b_emb_ref[...] + pe_ref[...]

    logits = jnp.dot(x, W_router_ref[...], preferred_element_type=jnp.float32)
    probs = jax.nn.softmax(logits, axis=-1)
    e_ids = jax.lax.broadcasted_iota(jnp.int32, (TILE, N_EXPERTS), 1)
    v1 = jnp.max(probs, axis=-1, keepdims=True)
    i1 = jnp.min(jnp.where(probs == v1, e_ids, N_EXPERTS), axis=-1,
                 keepdims=True)
    m1 = e_ids == i1
    probs2 = jnp.where(m1, -1.0, probs)
    v2 = jnp.max(probs2, axis=-1, keepdims=True)
    i2 = jnp.min(jnp.where(probs2 == v2, e_ids, N_EXPERTS), axis=-1,
                 keepdims=True)
    m2 = e_ids == i2
    denom = v1 + v2
    gates = jnp.where(m1, v1 / denom, 0.0) + jnp.where(m2, v2 / denom, 0.0)

    xb = x.astype(jnp.bfloat16)
    cc = jnp.bfloat16(0.7978845608028654)
    ca = jnp.bfloat16(0.035677408136300153)
    gb = (0.5 * gates).astype(jnp.bfloat16)
    acc = jnp.dot(gates, b2_ref[...], preferred_element_type=jnp.float32)
    NCHUNK = 2
    EPC = N_EXPERTS // NCHUNK
    W = EPC * D_FF
    for c in range(NCHUNK):
        sl = pl.ds(c * W, W)
        hb = jnp.dot(xb, W1c_ref[:, sl],
                     preferred_element_type=jnp.float32).astype(jnp.bfloat16)
        hb = hb + b1c_ref[sl]
        u2 = hb * hb
        z = hb * (u2 * ca + cc)
        t = jnp.tanh(z)
        Hb = hb + hb * t
        Hg = jnp.concatenate(
            [Hb[:, e * D_FF:(e + 1) * D_FF] * gb[:, c * EPC + e:c * EPC + e + 1]
             for e in range(EPC)], axis=1)
        acc = acc + jnp.dot(Hg, W2c_ref[sl, :],
                            preferred_element_type=jnp.float32)

    tokens = x + acc
    y_ref[...] = jnp.dot(tokens, W_head_ref[...],
                         preferred_element_type=jnp.float32) + b_head_ref[...]


@jax.jit
def kernel(z, W_emb, b_emb, W_router, W1, b1, W2, b2, W_head, b_head):
    bs, num_patch, n_vars, patch_len = z.shape
    d_model = W_emb.shape[1]
    zt = z.transpose(0, 2, 1, 3).reshape(TOKENS, patch_len)
    pe = _sincos_pos(num_patch, d_model)
    pe_tile = jnp.asarray(np.tile(pe, (TILE // num_patch, 1)))

    grid = (TOKENS // TILE,)
    y_flat = pl.pallas_call(
        _moe_block,
        grid=grid,
        in_specs=[
            pl.BlockSpec((TILE, patch_len), lambda i: (i, 0)),
            pl.BlockSpec((TILE, d_model), lambda i: (0, 0)),
            pl.BlockSpec((patch_len, d_model), lambda i: (0, 0)),
            pl.BlockSpec((d_model,), lambda i: (0,)),
            pl.BlockSpec((d_model, N_EXPERTS), lambda i: (0, 0)),
            pl.BlockSpec((d_model, N_EXPERTS * D_FF), lambda i: (0, 0)),
            pl.BlockSpec((N_EXPERTS * D_FF,), lambda i: (0,)),
            pl.BlockSpec((N_EXPERTS * D_FF, d_model), lambda i: (0, 0)),
            pl.BlockSpec((N_EXPERTS, d_model), lambda i: (0, 0)),
            pl.BlockSpec((d_model, patch_len), lambda i: (0, 0)),
            pl.BlockSpec((patch_len,), lambda i: (0,)),
        ],
        out_specs=pl.BlockSpec((TILE, patch_len), lambda i: (i, 0)),
        out_shape=jax.ShapeDtypeStruct((TOKENS, patch_len), jnp.float32),
        compiler_params=pltpu.CompilerParams(
            dimension_semantics=("arbitrary",)),
    )(zt, pe_tile, W_emb, b_emb, W_router,
      W1.transpose(1, 0, 2).reshape(d_model, N_EXPERTS * D_FF)
        .astype(jnp.bfloat16),
      b1.reshape(N_EXPERTS * D_FF).astype(jnp.bfloat16),
      W2.reshape(N_EXPERTS * D_FF, d_model).astype(jnp.bfloat16),
      b2, W_head, b_head)

    y = y_flat.reshape(bs, n_vars, num_patch, patch_len)
    y = y.transpose(0, 2, 3, 1).reshape(bs, num_patch * patch_len, n_vars)
    return y
